# Initial kernel scaffold; baseline (speedup 1.0000x reference)
import jax
import jax.numpy as jnp
from jax.experimental import pallas as pl


def kernel(x, adj, coords, U, b):
    raise NotImplementedError("write your pallas kernel here")


# speedup vs baseline: 136.2096x; 136.2096x over previous
import functools

import jax
import jax.numpy as jnp
from jax.experimental import pallas as pl
from jax.experimental.pallas import tpu as pltpu

N = 4096
OUT_F = 16
ROW_BLOCK = 128


def _body(adj_ref, crows_ref, cT_ref, UT_ref, U_ref, b_ref, out_ref):
    m = (adj_ref[:] > 0).astype(jnp.float32)
    cnt = jnp.sum(m, axis=1, keepdims=True)
    pT = jnp.dot(UT_ref[:], cT_ref[:], preferred_element_type=jnp.float32)
    q = jnp.dot(crows_ref[:], U_ref[:], preferred_element_type=jnp.float32) - b_ref[:]
    cols = []
    for f in range(OUT_F):
        pj = pT[f, :][None, :]
        qf = q[:, f][:, None]
        contrib = jnp.maximum(pj - qf, 0.0) * m
        cols.append(jnp.sum(contrib, axis=1, keepdims=True))
    acc = jnp.concatenate(cols, axis=1)
    mean = acc / jnp.maximum(cnt, 1.0)
    out_ref[:] = jnp.where(cnt > 0, mean, 0.0)


@jax.jit
def kernel(x, adj, coords, U, b):
    del x
    coords = coords.astype(jnp.float32)
    cT = coords.T
    UT = U.T
    b2 = b.reshape(1, OUT_F)
    grid = (N // ROW_BLOCK,)
    return pl.pallas_call(
        _body,
        grid=grid,
        in_specs=[
            pl.BlockSpec((ROW_BLOCK, N), lambda i: (i, 0)),
            pl.BlockSpec((ROW_BLOCK, 2), lambda i: (i, 0)),
            pl.BlockSpec((2, N), lambda i: (0, 0)),
            pl.BlockSpec((OUT_F, 2), lambda i: (0, 0)),
            pl.BlockSpec((2, OUT_F), lambda i: (0, 0)),
            pl.BlockSpec((1, OUT_F), lambda i: (0, 0)),
        ],
        out_specs=pl.BlockSpec((ROW_BLOCK, OUT_F), lambda i: (i, 0)),
        out_shape=jax.ShapeDtypeStruct((N, OUT_F), jnp.float32),
        compiler_params=pltpu.CompilerParams(
            dimension_semantics=("parallel",),
        ),
    )(adj, coords, cT, UT, U, b2)
